# baseline (speedup 1.0000x reference)
import functools

import numpy as np

import jax
import jax.numpy as jnp
from jax import lax
from jax.experimental import pallas as pl
from jax.experimental.pallas import tpu as pltpu
from jax.experimental.pallas import tpu_sc as plsc

B = 16384
F = 26
V = 100000
D = 64
NF = F + 1
NPAIR = NF * (NF - 1) // 2

_NC, _NS = 2, 16
_NW = _NC * _NS
_FL = 4096
_UNROLL = 8


@functools.cache
def _make_sc_gather():
    mesh = plsc.VectorSubcoreMesh(core_axis_name="c", subcore_axis_name="s")

    @functools.partial(
        pl.kernel,
        mesh=mesh,
        out_type=jax.ShapeDtypeStruct((F * D, B), jnp.float32),
        scratch_types=[
            pltpu.VMEM((B,), jnp.int32),
            pltpu.VMEM((V,), jnp.float32),
            pltpu.VMEM((2, _FL), jnp.float32),
            pltpu.SemaphoreType.DMA,
            pltpu.SemaphoreType.DMA,
            pltpu.SemaphoreType.DMA,
        ],
        compiler_params=pltpu.CompilerParams(needs_layout_passes=False),
    )
    def _sc_gather(tab_hbm, idx_hbm, out_hbm, idx_v, row_v, gat_v, sem, osem, isem):
        wid = lax.axis_index("s") * _NC + lax.axis_index("c")

        def row_body(j, carry):
            f = j // 2
            dd = (j % 2) * _NW
            r = f * D + wid + dd

            icp = pltpu.async_copy(idx_hbm.at[f], idx_v, isem)
            rcp = pltpu.async_copy(tab_hbm.at[r], row_v, sem)
            icp.wait()
            rcp.wait()
            prev = None
            for part in range(B // _FL):
                buf = part % 2

                @plsc.parallel_loop(0, _FL, step=16, unroll=_UNROLL)
                def grp(o, part=part, buf=buf):
                    idxs = idx_v[pl.ds(part * _FL + o, 16)]
                    gat_v[buf, pl.ds(o, 16)] = plsc.load_gather(row_v, [idxs])

                if prev is not None:
                    prev.wait()
                prev = pltpu.async_copy(
                    gat_v.at[buf], out_hbm.at[r, pl.ds(part * _FL, _FL)], osem)
            prev.wait()
            return carry

        lax.fori_loop(0, 2 * F, row_body, 0)

    return _sc_gather


_BT = 1024
_G = B // _BT


def _tc_body(dxT_ref, embsT_ref, w1t_ref, b1_ref, w2t_ref, b2_ref, w3t_ref,
             b3_ref, w4t_ref, b4_ref, w5t_ref, b5_ref, out_ref, xf_s):
    f32 = jnp.float32
    dxT = dxT_ref[...]
    h = jnp.maximum(
        jnp.dot(w1t_ref[...], dxT, preferred_element_type=f32) + b1_ref[...], 0.0)
    xbT = jnp.maximum(
        jnp.dot(w2t_ref[...], h, preferred_element_type=f32) + b2_ref[...], 0.0)
    xf_s[0:D, :] = xbT
    q = D
    for k in range(1, NF):
        for n in range(NF - k):
            m = n + k
            a = xbT if n == 0 else embsT_ref[(n - 1) * D:n * D, :]
            b = embsT_ref[(m - 1) * D:m * D, :]
            xf_s[q, :] = jnp.sum(a * b, axis=0)
            q += 1
    xf = xf_s[0:D + NPAIR, :]
    t = jnp.maximum(
        jnp.dot(w3t_ref[...], xf, preferred_element_type=f32) + b3_ref[...], 0.0)
    t = jnp.maximum(
        jnp.dot(w4t_ref[...], t, preferred_element_type=f32) + b4_ref[...], 0.0)
    z = jnp.dot(w5t_ref[...], t, preferred_element_type=f32) + b5_ref[...]
    out_ref[0] = 1.0 / (1.0 + jnp.exp(-z))


def _pair_perm() -> np.ndarray:
    rows, cols = np.triu_indices(NF, k=1)
    triu_pos = {(int(r), int(c)): i for i, (r, c) in enumerate(zip(rows, cols))}
    perm = [triu_pos[(n, n + k)] for k in range(1, NF) for n in range(NF - k)]
    return np.asarray(perm, dtype=np.int32)


_PERM = _pair_perm()


def kernel(dense_x, sparse_x, emb_tables, W1, b1, W2, b2, W3, b3, W4, b4, W5, b5):
    tabP = jnp.transpose(emb_tables, (0, 2, 1)).reshape(F * D, V)
    idxT = sparse_x.T.astype(jnp.int32)
    embsT = _make_sc_gather()(tabP, idxT)

    dxT = dense_x.T
    w3p = jnp.concatenate([W3[:D], W3[D:][_PERM]], axis=0)
    args = (
        dxT, embsT,
        W1.T, b1.reshape(-1, 1), W2.T, b2.reshape(-1, 1),
        w3p.T, b3.reshape(-1, 1), W4.T, b4.reshape(-1, 1),
        W5.T, b5.reshape(-1, 1),
    )
    full = lambda i: (0, 0)
    out = pl.pallas_call(
        _tc_body,
        grid=(_G,),
        in_specs=[
            pl.BlockSpec((13, _BT), lambda i: (0, i)),
            pl.BlockSpec((F * D, _BT), lambda i: (0, i)),
            pl.BlockSpec((256, 13), full), pl.BlockSpec((256, 1), full),
            pl.BlockSpec((D, 256), full), pl.BlockSpec((D, 1), full),
            pl.BlockSpec((128, D + NPAIR), full), pl.BlockSpec((128, 1), full),
            pl.BlockSpec((D, 128), full), pl.BlockSpec((D, 1), full),
            pl.BlockSpec((1, D), full), pl.BlockSpec((1, 1), full),
        ],
        out_specs=pl.BlockSpec((1, 1, _BT), lambda i: (i, 0, 0)),
        out_shape=jax.ShapeDtypeStruct((_G, 1, _BT), jnp.float32),
        scratch_shapes=[pltpu.VMEM((D + NPAIR + 1, _BT), jnp.float32)],
        compiler_params=pltpu.CompilerParams(
            dimension_semantics=("arbitrary",),
        ),
    )(*args)
    return out.reshape(B, 1)


# speedup vs baseline: 1.0727x; 1.0727x over previous
import functools

import numpy as np

import jax
import jax.numpy as jnp
from jax import lax
from jax.experimental import pallas as pl
from jax.experimental.pallas import tpu as pltpu
from jax.experimental.pallas import tpu_sc as plsc

B = 16384
F = 26
V = 100000
D = 64
NF = F + 1
NPAIR = NF * (NF - 1) // 2

_NC, _NS = 2, 16
_NW = _NC * _NS
_FL = 4096
_UNROLL = 8


@functools.cache
def _make_sc_gather():
    mesh = plsc.VectorSubcoreMesh(core_axis_name="c", subcore_axis_name="s")

    @functools.partial(
        pl.kernel,
        mesh=mesh,
        out_type=jax.ShapeDtypeStruct((F * D, B), jnp.float32),
        scratch_types=[
            pltpu.VMEM((B,), jnp.int32),
            pltpu.VMEM((V,), jnp.float32),
            pltpu.VMEM((2, _FL), jnp.float32),
            pltpu.SemaphoreType.DMA,
            pltpu.SemaphoreType.DMA,
            pltpu.SemaphoreType.DMA,
        ],
        compiler_params=pltpu.CompilerParams(needs_layout_passes=False),
    )
    def _sc_gather(tab_hbm, idx_hbm, out_hbm, idx_v, row_v, gat_v, sem, osem, isem):
        wid = lax.axis_index("s") * _NC + lax.axis_index("c")

        def row_body(j, carry):
            f = j // 2
            dd = (j % 2) * _NW
            r = f * D + wid + dd

            icp = pltpu.async_copy(idx_hbm.at[f], idx_v, isem)
            rcp = pltpu.async_copy(tab_hbm.at[r], row_v, sem)
            icp.wait()
            rcp.wait()
            prev = None
            for part in range(B // _FL):
                buf = part % 2

                @plsc.parallel_loop(0, _FL, step=16, unroll=_UNROLL)
                def grp(o, part=part, buf=buf):
                    idxs = idx_v[pl.ds(part * _FL + o, 16)]
                    gat_v[buf, pl.ds(o, 16)] = plsc.load_gather(row_v, [idxs])

                if prev is not None:
                    prev.wait()
                prev = pltpu.async_copy(
                    gat_v.at[buf], out_hbm.at[r, pl.ds(part * _FL, _FL)], osem)
            prev.wait()
            return carry

        lax.fori_loop(0, 2 * F, row_body, 0)

    return _sc_gather


_BT = 512
_G = B // _BT


def _tc_body(dxT_ref, embsT_ref, w1t_ref, b1_ref, w2t_ref, b2_ref, w3t_ref,
             b3_ref, w4t_ref, b4_ref, w5t_ref, b5_ref, out_ref, xf_s):
    f32 = jnp.float32
    dxT = dxT_ref[...]
    h = jnp.maximum(
        jnp.dot(w1t_ref[...], dxT, preferred_element_type=f32) + b1_ref[...], 0.0)
    xbT = jnp.maximum(
        jnp.dot(w2t_ref[...], h, preferred_element_type=f32) + b2_ref[...], 0.0)
    xf_s[0:D, :] = xbT
    q = D
    for k in range(1, NF):
        for n in range(NF - k):
            m = n + k
            a = xbT if n == 0 else embsT_ref[(n - 1) * D:n * D, :]
            b = embsT_ref[(m - 1) * D:m * D, :]
            xf_s[q, :] = jnp.sum(a * b, axis=0)
            q += 1
    xf = xf_s[0:D + NPAIR, :]
    t = jnp.maximum(
        jnp.dot(w3t_ref[...], xf, preferred_element_type=f32) + b3_ref[...], 0.0)
    t = jnp.maximum(
        jnp.dot(w4t_ref[...], t, preferred_element_type=f32) + b4_ref[...], 0.0)
    z = jnp.dot(w5t_ref[...], t, preferred_element_type=f32) + b5_ref[...]
    out_ref[0] = 1.0 / (1.0 + jnp.exp(-z))


def _pair_perm() -> np.ndarray:
    rows, cols = np.triu_indices(NF, k=1)
    triu_pos = {(int(r), int(c)): i for i, (r, c) in enumerate(zip(rows, cols))}
    perm = [triu_pos[(n, n + k)] for k in range(1, NF) for n in range(NF - k)]
    return np.asarray(perm, dtype=np.int32)


_PERM = _pair_perm()


def kernel(dense_x, sparse_x, emb_tables, W1, b1, W2, b2, W3, b3, W4, b4, W5, b5):
    tabP = jnp.transpose(emb_tables, (0, 2, 1)).reshape(F * D, V)
    idxT = sparse_x.T.astype(jnp.int32)
    embsT = _make_sc_gather()(tabP, idxT)

    dxT = dense_x.T
    w3p = jnp.concatenate([W3[:D], W3[D:][_PERM]], axis=0)
    args = (
        dxT, embsT,
        W1.T, b1.reshape(-1, 1), W2.T, b2.reshape(-1, 1),
        w3p.T, b3.reshape(-1, 1), W4.T, b4.reshape(-1, 1),
        W5.T, b5.reshape(-1, 1),
    )
    full = lambda i: (0, 0)
    out = pl.pallas_call(
        _tc_body,
        grid=(_G,),
        in_specs=[
            pl.BlockSpec((13, _BT), lambda i: (0, i)),
            pl.BlockSpec((F * D, _BT), lambda i: (0, i)),
            pl.BlockSpec((256, 13), full), pl.BlockSpec((256, 1), full),
            pl.BlockSpec((D, 256), full), pl.BlockSpec((D, 1), full),
            pl.BlockSpec((128, D + NPAIR), full), pl.BlockSpec((128, 1), full),
            pl.BlockSpec((D, 128), full), pl.BlockSpec((D, 1), full),
            pl.BlockSpec((1, D), full), pl.BlockSpec((1, 1), full),
        ],
        out_specs=pl.BlockSpec((1, 1, _BT), lambda i: (i, 0, 0)),
        out_shape=jax.ShapeDtypeStruct((_G, 1, _BT), jnp.float32),
        scratch_shapes=[pltpu.VMEM((D + NPAIR + 1, _BT), jnp.float32)],
        compiler_params=pltpu.CompilerParams(
            dimension_semantics=("arbitrary",),
        ),
    )(*args)
    return out.reshape(B, 1)


# speedup vs baseline: 1.1557x; 1.0773x over previous
import functools

import numpy as np

import jax
import jax.numpy as jnp
from jax import lax
from jax.experimental import pallas as pl
from jax.experimental.pallas import tpu as pltpu
from jax.experimental.pallas import tpu_sc as plsc

B = 16384
F = 26
V = 100000
D = 64
NF = F + 1
NPAIR = NF * (NF - 1) // 2

_NC, _NS = 2, 16
_NW = _NC * _NS
_FL = 4096
_UNROLL = 8


@functools.cache
def _make_sc_gather():
    mesh = plsc.VectorSubcoreMesh(core_axis_name="c", subcore_axis_name="s")

    @functools.partial(
        pl.kernel,
        mesh=mesh,
        out_type=jax.ShapeDtypeStruct((F * D, B), jnp.float32),
        scratch_types=[
            pltpu.VMEM((B,), jnp.int32),
            pltpu.VMEM((V,), jnp.float32),
            pltpu.VMEM((2, _FL), jnp.float32),
            pltpu.SemaphoreType.DMA,
            pltpu.SemaphoreType.DMA,
            pltpu.SemaphoreType.DMA,
        ],
        compiler_params=pltpu.CompilerParams(needs_layout_passes=False),
    )
    def _sc_gather(tab_hbm, idx_hbm, out_hbm, idx_v, row_v, gat_v, sem, osem, isem):
        wid = lax.axis_index("s") * _NC + lax.axis_index("c")

        def row_body(j, carry):
            f = j // 2
            dd = (j % 2) * _NW
            r = f * D + wid + dd

            @pl.when(dd == 0)
            def _():
                icp = pltpu.async_copy(idx_hbm.at[f], idx_v, isem)
                rcp = pltpu.async_copy(tab_hbm.at[r], row_v, sem)
                icp.wait()
                rcp.wait()

            @pl.when(dd != 0)
            def _():
                pltpu.async_copy(tab_hbm.at[r], row_v, sem).wait()
            prev = None
            for part in range(B // _FL):
                buf = part % 2

                @plsc.parallel_loop(0, _FL, step=16, unroll=_UNROLL)
                def grp(o, part=part, buf=buf):
                    idxs = idx_v[pl.ds(part * _FL + o, 16)]
                    gat_v[buf, pl.ds(o, 16)] = plsc.load_gather(row_v, [idxs])

                if prev is not None:
                    prev.wait()
                prev = pltpu.async_copy(
                    gat_v.at[buf], out_hbm.at[r, pl.ds(part * _FL, _FL)], osem)
            prev.wait()
            return carry

        lax.fori_loop(0, 2 * F, row_body, 0)

    return _sc_gather


_BT = 512
_G = B // _BT


def _tc_body(dxT_ref, embsT_ref, w1t_ref, b1_ref, w2t_ref, b2_ref, w3t_ref,
             b3_ref, w4t_ref, b4_ref, w5t_ref, b5_ref, out_ref, xf_s):
    f32 = jnp.float32
    dxT = dxT_ref[...]
    h = jnp.maximum(
        jnp.dot(w1t_ref[...], dxT, preferred_element_type=f32) + b1_ref[...], 0.0)
    xbT = jnp.maximum(
        jnp.dot(w2t_ref[...], h, preferred_element_type=f32) + b2_ref[...], 0.0)
    xf_s[0:D, :] = xbT
    q = D
    for k in range(1, NF):
        for n in range(NF - k):
            m = n + k
            a = xbT if n == 0 else embsT_ref[(n - 1) * D:n * D, :]
            b = embsT_ref[(m - 1) * D:m * D, :]
            xf_s[q, :] = jnp.sum(a * b, axis=0)
            q += 1
    xf = xf_s[0:D + NPAIR, :]
    t = jnp.maximum(
        jnp.dot(w3t_ref[...], xf, preferred_element_type=f32) + b3_ref[...], 0.0)
    t = jnp.maximum(
        jnp.dot(w4t_ref[...], t, preferred_element_type=f32) + b4_ref[...], 0.0)
    z = jnp.dot(w5t_ref[...], t, preferred_element_type=f32) + b5_ref[...]
    out_ref[0] = 1.0 / (1.0 + jnp.exp(-z))


def _pair_perm() -> np.ndarray:
    rows, cols = np.triu_indices(NF, k=1)
    triu_pos = {(int(r), int(c)): i for i, (r, c) in enumerate(zip(rows, cols))}
    perm = [triu_pos[(n, n + k)] for k in range(1, NF) for n in range(NF - k)]
    return np.asarray(perm, dtype=np.int32)


_PERM = _pair_perm()


def kernel(dense_x, sparse_x, emb_tables, W1, b1, W2, b2, W3, b3, W4, b4, W5, b5):
    tabP = jnp.transpose(emb_tables, (0, 2, 1)).reshape(F * D, V)
    idxT = sparse_x.T.astype(jnp.int32)
    embsT = _make_sc_gather()(tabP, idxT)

    dxT = dense_x.T
    w3p = jnp.concatenate([W3[:D], W3[D:][_PERM]], axis=0)
    args = (
        dxT, embsT,
        W1.T, b1.reshape(-1, 1), W2.T, b2.reshape(-1, 1),
        w3p.T, b3.reshape(-1, 1), W4.T, b4.reshape(-1, 1),
        W5.T, b5.reshape(-1, 1),
    )
    full = lambda i: (0, 0)
    out = pl.pallas_call(
        _tc_body,
        grid=(_G,),
        in_specs=[
            pl.BlockSpec((13, _BT), lambda i: (0, i)),
            pl.BlockSpec((F * D, _BT), lambda i: (0, i)),
            pl.BlockSpec((256, 13), full), pl.BlockSpec((256, 1), full),
            pl.BlockSpec((D, 256), full), pl.BlockSpec((D, 1), full),
            pl.BlockSpec((128, D + NPAIR), full), pl.BlockSpec((128, 1), full),
            pl.BlockSpec((D, 128), full), pl.BlockSpec((D, 1), full),
            pl.BlockSpec((1, D), full), pl.BlockSpec((1, 1), full),
        ],
        out_specs=pl.BlockSpec((1, 1, _BT), lambda i: (i, 0, 0)),
        out_shape=jax.ShapeDtypeStruct((_G, 1, _BT), jnp.float32),
        scratch_shapes=[pltpu.VMEM((D + NPAIR + 1, _BT), jnp.float32)],
        compiler_params=pltpu.CompilerParams(
            dimension_semantics=("arbitrary",),
        ),
    )(*args)
    return out.reshape(B, 1)
